# baseline (speedup 1.0000x reference)
import functools

import jax
import jax.numpy as jnp
import numpy as np
from jax import lax
from jax.experimental import pallas as pl
from jax.experimental.pallas import tpu as pltpu
from jax.experimental.pallas import tpu_sc as plsc

C = 384
H = W = 384
HW = H * W
NM = 1024
NNM = NM * 150
K = NM + NNM
NC = 2
NS = 16
NW = NC * NS
PB = 4864
K_PAD = NW * PB
CH = 128
NCH = PB // CH
DROWS = CH * 16 // 128


def _pack_pair(lo, hi):
    p = pltpu.pack_elementwise([lo, hi], packed_dtype=jnp.bfloat16)
    return lax.bitcast_convert_type(p, jnp.float32)


def _tr1_body(alo_ref, ahi_ref, blo_ref, bhi_ref, a0, b0):
    for lo_ref, hi_ref, out in ((alo_ref, ahi_ref, a0), (blo_ref, bhi_ref, b0)):
        for h in range(8):
            out[h * 128:(h + 1) * 128, :] = _pack_pair(
                lo_ref[0, :, h, :], hi_ref[0, :, h, :]
            ).T


def _tr2_body(a_ref, b_ref, ab1):
    for h in range(8):
        ab1[h * 128:(h + 1) * 128, :] = _pack_pair(
            a_ref[0, :, h, :], b_ref[0, :, h, :]
        ).T


def _img_spec(cblk):
    return pl.BlockSpec((1, 128, 8, 128), lambda i, c=cblk: (0, c, i // 3, i % 3))


_out_spec = pl.BlockSpec((1024, 128), lambda i: (i, 0))
_GRID = ((H // 8) * (W // 128),)

_transpose1 = pl.pallas_call(
    _tr1_body,
    grid=_GRID,
    in_specs=[_img_spec(0), _img_spec(1), _img_spec(0), _img_spec(1)],
    out_specs=[_out_spec] * 2,
    out_shape=[jax.ShapeDtypeStruct((HW, 128), jnp.float32)] * 2,
)

_transpose2 = pl.pallas_call(
    _tr2_body,
    grid=_GRID,
    in_specs=[_img_spec(2), _img_spec(2)],
    out_specs=_out_spec,
    out_shape=jax.ShapeDtypeStruct((HW, 128), jnp.float32),
)

_mesh = plsc.VectorSubcoreMesh(
    core_axis_name="c", subcore_axis_name="s", num_cores=NC, num_subcores=NS
)
_sc_params = pltpu.CompilerParams(
    use_tc_tiling_on_sc=False, needs_layout_passes=False
)


def _wid_base():
    wid = lax.axis_index("s") * NC + lax.axis_index("c")
    return wid, wid * PB


def _pair_partial(a_bufs, b_bufs, par, r, init):
    accs = [init, jnp.zeros((16,), jnp.float32),
            jnp.zeros((16,), jnp.float32), jnp.zeros((16,), jnp.float32)]
    cnt = 0
    for abuf, bbuf in zip(a_bufs, b_bufs):
        for j in range(8):
            va = abuf[par, r, pl.ds(16 * j, 16)]
            vb = bbuf[par, r, pl.ds(16 * j, 16)]
            d = plsc.bitcast(va, jnp.bfloat16) - plsc.bitcast(vb, jnp.bfloat16)
            dl, dh = plsc.unpack(d, format=plsc.PackFormat.INTERLEAVED)
            accs[cnt % 2] = accs[cnt % 2] + dl * dl
            accs[2 + cnt % 2] = accs[2 + cnt % 2] + dh * dh
            cnt += 1
    return (accs[0] + accs[1]) + (accs[2] + accs[3])


@functools.partial(
    pl.kernel,
    out_type=jax.ShapeDtypeStruct((K_PAD * 16 // 128, 128), jnp.float32),
    mesh=_mesh,
    scratch_types=[
        pltpu.VMEM((PB,), jnp.int32),
        pltpu.VMEM((PB,), jnp.int32),
        pltpu.VMEM((2, CH, 128), jnp.float32),
        pltpu.VMEM((2, CH, 128), jnp.float32),
        pltpu.VMEM((2, DROWS, 128), jnp.float32),
        pltpu.SemaphoreType.DMA,
        pltpu.SemaphoreType.DMA,
        pltpu.SemaphoreType.DMA,
        pltpu.SemaphoreType.DMA,
    ],
    compiler_params=_sc_params,
)
def _sc_phase1(a0_h, b0_h, ia_hbm, ib_hbm, d_hbm,
               ia_v, ib_v, a_v, b_v, d_v, sem0, sem1, dsem0, dsem1):
    _, base = _wid_base()
    pltpu.sync_copy(ia_hbm.at[pl.ds(base, PB)], ia_v)
    pltpu.sync_copy(ib_hbm.at[pl.ds(base, PB)], ib_v)
    sems = (sem0, sem1)
    dsems = (dsem0, dsem1)
    zero = jnp.zeros((16,), jnp.float32)

    def fire(ch, par):
        c0 = ch * CH
        pltpu.async_copy(a0_h.at[ia_v.at[pl.ds(c0, CH)]], a_v.at[par], sems[par])
        pltpu.async_copy(b0_h.at[ib_v.at[pl.ds(c0, CH)]], b_v.at[par], sems[par])

    def drain(par):
        for tab, buf in ((a0_h, a_v), (b0_h, b_v)):
            pltpu.make_async_copy(
                tab.at[pl.ds(0, CH)], buf.at[par], sems[par]
            ).wait()

    def dwait(par):
        pltpu.make_async_copy(
            d_v.at[par], d_hbm.at[pl.ds(0, DROWS)], dsems[par]
        ).wait()

    def compute(ch, par):
        def pair_body(r, _):
            acc = _pair_partial((a_v,), (b_v,), par, r, zero)
            d_v[par, r >> 3, pl.ds((r & 7) * 16, 16)] = acc
            return 0

        lax.fori_loop(0, CH, pair_body, 0, unroll=2)
        drow = (base + ch * CH) >> 3
        pltpu.async_copy(d_v.at[par], d_hbm.at[pl.ds(drow, DROWS)], dsems[par])

    fire(0, 0)

    def body(i, _):
        ch0 = 2 * i
        fire(ch0 + 1, 1)
        drain(0)

        @pl.when(i > 0)
        def _():
            dwait(0)

        compute(ch0, 0)

        @pl.when(i < NCH // 2 - 1)
        def _():
            fire(ch0 + 2, 0)

        drain(1)

        @pl.when(i > 0)
        def _():
            dwait(1)

        compute(ch0 + 1, 1)
        return 0

    lax.fori_loop(0, NCH // 2, body, 0)
    dwait(0)
    dwait(1)


@functools.partial(
    pl.kernel,
    out_type=jax.ShapeDtypeStruct((NW, 2, 16), jnp.float32),
    mesh=_mesh,
    scratch_types=[
        pltpu.VMEM((PB,), jnp.int32),
        pltpu.VMEM((PB,), jnp.int32),
        pltpu.VMEM((2, CH, 128), jnp.float32),
        pltpu.VMEM((2, CH, 128), jnp.float32),
        pltpu.VMEM((2, DROWS, 128), jnp.float32),
        pltpu.VMEM((2, 16), jnp.float32),
        pltpu.SemaphoreType.DMA,
        pltpu.SemaphoreType.DMA,
    ],
    compiler_params=_sc_params,
)
def _sc_phase2(ab1_h, ia_hbm, ib_hbm, d_hbm, out_hbm,
               ia_v, ib_v, a_v, b_v, d_v, acc_v, sem0, sem1):
    wid, base = _wid_base()
    pltpu.sync_copy(ia_hbm.at[pl.ds(base, PB)], ia_v)
    pltpu.sync_copy(ib_hbm.at[pl.ds(base, PB)], ib_v)
    sems = (sem0, sem1)
    zero = jnp.zeros((16,), jnp.float32)

    def fire(ch, par):
        c0 = ch * CH
        pltpu.async_copy(ab1_h.at[ia_v.at[pl.ds(c0, CH)]], a_v.at[par], sems[par])
        pltpu.async_copy(ab1_h.at[ib_v.at[pl.ds(c0, CH)]], b_v.at[par], sems[par])
        pltpu.async_copy(
            d_hbm.at[pl.ds((base + c0) >> 3, DROWS)], d_v.at[par], sems[par]
        )

    def drain(par):
        for buf in (a_v, b_v):
            pltpu.make_async_copy(
                ab1_h.at[pl.ds(0, CH)], buf.at[par], sems[par]
            ).wait()
        pltpu.make_async_copy(
            d_hbm.at[pl.ds(0, DROWS)], d_v.at[par], sems[par]
        ).wait()

    def compute(ch, par, carry):
        gbase = base + ch * CH

        def pair_body(r, c2):
            m, n = c2
            accs = [d_v[par, r >> 3, pl.ds((r & 7) * 16, 16)],
                    zero, zero, zero]
            for j in range(8):
                va = a_v[par, r, pl.ds(16 * j, 16)]
                vb = b_v[par, r, pl.ds(16 * j, 16)]
                va_lo, _ = plsc.unpack(
                    plsc.bitcast(va, jnp.bfloat16),
                    format=plsc.PackFormat.INTERLEAVED,
                )
                _, vb_hi = plsc.unpack(
                    plsc.bitcast(vb, jnp.bfloat16),
                    format=plsc.PackFormat.INTERLEAVED,
                )
                d = va_lo - vb_hi
                accs[j % 4] = accs[j % 4] + d * d
            acc = (accs[0] + accs[1]) + (accs[2] + accs[3])
            s = plsc.cumsum(acc)
            gv = jnp.full((16,), gbase + r, jnp.int32)
            is_m = gv < NM
            ok = gv < K
            m = m + jnp.where(is_m, s, 0.0)
            n = n + jnp.where(
                jnp.logical_and(ok, jnp.logical_not(is_m)),
                jnp.maximum(0.5 - s, 0.0),
                0.0,
            )
            return m, n

        return lax.fori_loop(0, CH, pair_body, carry, unroll=2)

    fire(0, 0)

    def body(i, carry):
        ch0 = 2 * i
        fire(ch0 + 1, 1)
        drain(0)
        carry = compute(ch0, 0, carry)

        @pl.when(i < NCH // 2 - 1)
        def _():
            fire(ch0 + 2, 0)

        drain(1)
        carry = compute(ch0 + 1, 1, carry)
        return carry

    m_acc, n_acc = lax.fori_loop(0, NCH // 2, body, (zero, zero))
    acc_v[0] = m_acc
    acc_v[1] = n_acc
    pltpu.sync_copy(acc_v, out_hbm.at[wid])


def kernel(image_a_pred, image_b_pred, matches_a, matches_b,
           non_matches_a, non_matches_b):
    def pix(rc):
        r = rc[:, 0].astype(jnp.int32)
        c = rc[:, 1].astype(jnp.int32)
        return (r >> 3) * 3072 + ((c >> 7) << 10) + ((r & 7) << 7) + (c & 127)

    pad = jnp.zeros((K_PAD - K,), jnp.int32)
    ia = jnp.concatenate([pix(matches_a), pix(non_matches_a), pad])
    ib = jnp.concatenate([pix(matches_b), pix(non_matches_b), pad])

    a0, b0 = _transpose1(image_a_pred, image_a_pred,
                         image_b_pred, image_b_pred)
    dpart = _sc_phase1(a0, b0, ia, ib)
    ab1 = _transpose2(image_a_pred, image_b_pred)
    out = _sc_phase2(ab1, ia, ib, dpart)

    match_loss = jnp.sum(out[:, 0, 15]) / NM
    non_match_loss = jnp.sum(out[:, 1, 15]) / NNM
    loss = match_loss + non_match_loss
    return (loss, match_loss, non_match_loss)


# speedup vs baseline: 1.0840x; 1.0840x over previous
import functools

import jax
import jax.numpy as jnp
import numpy as np
from jax import lax
from jax.experimental import pallas as pl
from jax.experimental.pallas import tpu as pltpu
from jax.experimental.pallas import tpu_sc as plsc

C = 384
H = W = 384
HW = H * W
NM = 1024
NNM = NM * 150
K = NM + NNM
NC = 2
NS = 16
NW = NC * NS
PB = 4864
K_PAD = NW * PB
CH = 64
NCH = PB // CH


def _pack_pair(lo, hi):
    p = pltpu.pack_elementwise([lo, hi], packed_dtype=jnp.bfloat16)
    return lax.bitcast_convert_type(p, jnp.float32)


def _tr_body(alo_ref, ahi_ref, a2_ref, blo_ref, bhi_ref, b2_ref, a0, b0, ab1):
    for h in range(8):
        sl = slice(h * 128, (h + 1) * 128)
        a0[sl, :] = _pack_pair(alo_ref[0, :, h, :], ahi_ref[0, :, h, :]).T
        b0[sl, :] = _pack_pair(blo_ref[0, :, h, :], bhi_ref[0, :, h, :]).T
        ab1[sl, :] = _pack_pair(a2_ref[0, :, h, :], b2_ref[0, :, h, :]).T


def _img_spec(cblk):
    return pl.BlockSpec((1, 128, 8, 128), lambda i, c=cblk: (0, c, i // 3, i % 3))


_transpose = pl.pallas_call(
    _tr_body,
    grid=((H // 8) * (W // 128),),
    in_specs=[_img_spec(0), _img_spec(1), _img_spec(2)] * 2,
    out_specs=[pl.BlockSpec((1024, 128), lambda i: (i, 0))] * 3,
    out_shape=[jax.ShapeDtypeStruct((HW, 128), jnp.float32)] * 3,
)

_mesh = plsc.VectorSubcoreMesh(
    core_axis_name="c", subcore_axis_name="s", num_cores=NC, num_subcores=NS
)


@functools.partial(
    pl.kernel,
    out_type=jax.ShapeDtypeStruct((NW, 2, 16), jnp.float32),
    mesh=_mesh,
    scratch_types=[
        pltpu.VMEM((PB,), jnp.int32),
        pltpu.VMEM((PB,), jnp.int32),
    ]
    + [pltpu.VMEM((2, CH, 128), jnp.float32)] * 4
    + [
        pltpu.VMEM((2, 16), jnp.float32),
        pltpu.SemaphoreType.DMA,
        pltpu.SemaphoreType.DMA,
    ],
    compiler_params=pltpu.CompilerParams(
        use_tc_tiling_on_sc=False, needs_layout_passes=False
    ),
)
def _sc_dist(a0_h, b0_h, ab1_h, ia_hbm, ib_hbm, out_hbm,
             ia_v, ib_v, a0_v, b0_v, a1_v, b1_v, acc_v, sem0, sem1):
    wid = lax.axis_index("s") * NC + lax.axis_index("c")
    base = wid * PB
    pltpu.sync_copy(ia_hbm.at[pl.ds(base, PB)], ia_v)
    pltpu.sync_copy(ib_hbm.at[pl.ds(base, PB)], ib_v)
    zero = jnp.zeros((16,), jnp.float32)
    sems = (sem0, sem1)

    def fire(ch, par):
        c0 = ch * CH
        ias = ia_v.at[pl.ds(c0, CH)]
        ibs = ib_v.at[pl.ds(c0, CH)]
        pltpu.async_copy(a0_h.at[ias], a0_v.at[par], sems[par])
        pltpu.async_copy(b0_h.at[ibs], b0_v.at[par], sems[par])
        pltpu.async_copy(ab1_h.at[ias], a1_v.at[par], sems[par])
        pltpu.async_copy(ab1_h.at[ibs], b1_v.at[par], sems[par])

    def drain(par):
        for buf in (a0_v, b0_v, a1_v, b1_v):
            pltpu.make_async_copy(
                a0_h.at[pl.ds(0, CH)], buf.at[par], sems[par]
            ).wait()

    def compute(ch, par, carry):
        gbase = base + ch * CH

        def pair_body(r, c2):
            m, n = c2
            accs = [zero, zero, zero, zero]
            for j in range(8):
                va = a0_v[par, r, pl.ds(16 * j, 16)]
                vb = b0_v[par, r, pl.ds(16 * j, 16)]
                d = plsc.bitcast(va, jnp.bfloat16) - plsc.bitcast(
                    vb, jnp.bfloat16
                )
                dl, dh = plsc.unpack(d, format=plsc.PackFormat.INTERLEAVED)
                accs[j % 2] = accs[j % 2] + dl * dl
                accs[2 + j % 2] = accs[2 + j % 2] + dh * dh
            for j in range(8):
                va = a1_v[par, r, pl.ds(16 * j, 16)]
                vb = b1_v[par, r, pl.ds(16 * j, 16)]
                va_lo, _ = plsc.unpack(
                    plsc.bitcast(va, jnp.bfloat16),
                    format=plsc.PackFormat.INTERLEAVED,
                )
                _, vb_hi = plsc.unpack(
                    plsc.bitcast(vb, jnp.bfloat16),
                    format=plsc.PackFormat.INTERLEAVED,
                )
                d = va_lo - vb_hi
                accs[j % 4] = accs[j % 4] + d * d
            acc = (accs[0] + accs[1]) + (accs[2] + accs[3])
            s = plsc.cumsum(acc)
            gv = jnp.full((16,), gbase + r, jnp.int32)
            is_m = gv < NM
            ok = gv < K
            m = m + jnp.where(is_m, s, 0.0)
            n = n + jnp.where(
                jnp.logical_and(ok, jnp.logical_not(is_m)),
                jnp.maximum(0.5 - s, 0.0),
                0.0,
            )
            return m, n

        return lax.fori_loop(0, CH, pair_body, carry, unroll=2)

    fire(0, 0)

    def body(i, carry):
        ch0 = 2 * i
        fire(ch0 + 1, 1)
        drain(0)
        carry = compute(ch0, 0, carry)

        @pl.when(i < NCH // 2 - 1)
        def _():
            fire(ch0 + 2, 0)

        drain(1)
        carry = compute(ch0 + 1, 1, carry)
        return carry

    m_acc, n_acc = lax.fori_loop(0, NCH // 2, body, (zero, zero))
    acc_v[0] = m_acc
    acc_v[1] = n_acc
    pltpu.sync_copy(acc_v, out_hbm.at[wid])


def kernel(image_a_pred, image_b_pred, matches_a, matches_b,
           non_matches_a, non_matches_b):
    def pix(rc):
        r = rc[:, 0].astype(jnp.int32)
        c = rc[:, 1].astype(jnp.int32)
        return (r >> 3) * 3072 + ((c >> 7) << 10) + ((r & 7) << 7) + (c & 127)

    pad = jnp.zeros((K_PAD - K,), jnp.int32)
    ia = jnp.concatenate([pix(matches_a), pix(non_matches_a), pad])
    ib = jnp.concatenate([pix(matches_b), pix(non_matches_b), pad])

    a0, b0, ab1 = _transpose(image_a_pred, image_a_pred, image_a_pred,
                             image_b_pred, image_b_pred, image_b_pred)
    out = _sc_dist(a0, b0, ab1, ia, ib)

    match_loss = jnp.sum(out[:, 0, 15]) / NM
    non_match_loss = jnp.sum(out[:, 1, 15]) / NNM
    loss = match_loss + non_match_loss
    return (loss, match_loss, non_match_loss)
